# Initial kernel scaffold; baseline (speedup 1.0000x reference)
import jax
import jax.numpy as jnp
from jax.experimental import pallas as pl


def kernel(x, edge_index, W1l, b1l, W1r, W2l, b2l, W2r):
    raise NotImplementedError("write your pallas kernel here")


# speedup vs baseline: 4.6836x; 4.6836x over previous
import jax
import jax.numpy as jnp
from jax import lax
from jax.experimental import pallas as pl
from jax.experimental.pallas import tpu as pltpu
from jax.experimental.pallas import tpu_sc as plsc

N_NODES = 10000
D_IN = 128
D_HID = 256
D_OUT = 128

CHUNK = 128
N_PAD = 10240


def _seg_sum_sc(table, src_p, dst_p):
    info = plsc.get_sparse_core_info()
    nc, ns = info.num_cores, info.num_subcores
    nw = nc * ns
    d = table.shape[1]
    e_pad = src_p.shape[0]
    assert e_pad % (nw * CHUNK) == 0
    chunks_per_worker = e_pad // (nw * CHUNK)
    epw = chunks_per_worker * CHUNK
    rows_per_sub = N_PAD // ns
    assert rows_per_sub % CHUNK == 0
    zcopies = rows_per_sub // CHUNK

    mesh = plsc.VectorSubcoreMesh(core_axis_name="c", subcore_axis_name="s")

    def body(table_hbm, src_hbm, dst_hbm, sum_out, src_v, dst_v, rows_v,
             acc_sh, sem):
        cid = lax.axis_index("c")
        sid = lax.axis_index("s")
        wid = cid * ns + sid
        zero16 = jnp.zeros((16,), jnp.float32)

        def fz(i, carry):
            for j in range(d // 16):
                rows_v[i, pl.ds(j * 16, 16)] = zero16
            return carry
        lax.fori_loop(0, CHUNK, fz, 0)
        row0 = sid * rows_per_sub
        for k in range(zcopies):
            pltpu.sync_copy(rows_v, acc_sh.at[pl.ds(row0 + k * CHUNK, CHUNK)])
        plsc.subcore_barrier()

        eb = wid * epw

        def step(g, carry):
            base = eb + g * CHUNK
            pltpu.sync_copy(src_hbm.at[pl.ds(base, CHUNK)], src_v)
            pltpu.sync_copy(dst_hbm.at[pl.ds(base, CHUNK)], dst_v)
            pltpu.async_copy(table_hbm.at[src_v], rows_v, sem).wait()
            pltpu.sync_copy(rows_v, acc_sh.at[dst_v], add=True)
            return carry
        lax.fori_loop(0, chunks_per_worker, step, 0)

        plsc.subcore_barrier()
        pltpu.sync_copy(acc_sh.at[pl.ds(row0, rows_per_sub)],
                        sum_out.at[pl.ds(cid * N_PAD + row0, rows_per_sub)])

    fn = pl.kernel(
        body,
        out_type=[jax.ShapeDtypeStruct((nc * N_PAD, d), jnp.float32)],
        mesh=mesh,
        scratch_types=[
            pltpu.VMEM((CHUNK,), jnp.int32),
            pltpu.VMEM((CHUNK,), jnp.int32),
            pltpu.VMEM((CHUNK, d), jnp.float32),
            pltpu.VMEM_SHARED((N_PAD, d), jnp.float32),
            pltpu.SemaphoreType.DMA,
        ],
    )
    return fn(table, src_p, dst_p)[0].reshape(nc, N_PAD, d)


def _seg_cnt_sc(dst_p):
    info = plsc.get_sparse_core_info()
    nc, ns = info.num_cores, info.num_subcores
    nw = nc * ns
    d = 128
    e_pad = dst_p.shape[0]
    assert e_pad % (nw * CHUNK) == 0
    chunks_per_worker = e_pad // (nw * CHUNK)
    epw = chunks_per_worker * CHUNK
    rows_per_sub = N_PAD // ns
    zcopies = rows_per_sub // CHUNK

    mesh = plsc.VectorSubcoreMesh(core_axis_name="c", subcore_axis_name="s")

    def body(dst_hbm, cnt_out, dst_v, ones_v, cnt_sh, sem):
        cid = lax.axis_index("c")
        sid = lax.axis_index("s")
        wid = cid * ns + sid
        zero16 = jnp.zeros((16,), jnp.float32)

        def fz(i, carry):
            for j in range(d // 16):
                ones_v[i, pl.ds(j * 16, 16)] = zero16
            return carry
        lax.fori_loop(0, CHUNK, fz, 0)
        row0 = sid * rows_per_sub
        for k in range(zcopies):
            pltpu.sync_copy(ones_v, cnt_sh.at[pl.ds(row0 + k * CHUNK, CHUNK)])
        one16 = jnp.full((16,), 1.0, jnp.float32)

        def fo(i, carry):
            for j in range(d // 16):
                ones_v[i, pl.ds(j * 16, 16)] = one16
            return carry
        lax.fori_loop(0, CHUNK, fo, 0)
        plsc.subcore_barrier()

        eb = wid * epw

        def step(g, carry):
            base = eb + g * CHUNK
            pltpu.sync_copy(dst_hbm.at[pl.ds(base, CHUNK)], dst_v)
            pltpu.sync_copy(ones_v, cnt_sh.at[dst_v], add=True)
            return carry
        lax.fori_loop(0, chunks_per_worker, step, 0)

        plsc.subcore_barrier()
        pltpu.sync_copy(cnt_sh.at[pl.ds(row0, rows_per_sub)],
                        cnt_out.at[pl.ds(cid * N_PAD + row0, rows_per_sub)])

    fn = pl.kernel(
        body,
        out_type=[jax.ShapeDtypeStruct((nc * N_PAD, d), jnp.float32)],
        mesh=mesh,
        scratch_types=[
            pltpu.VMEM((CHUNK,), jnp.int32),
            pltpu.VMEM((CHUNK, d), jnp.float32),
            pltpu.VMEM_SHARED((N_PAD, d), jnp.float32),
            pltpu.SemaphoreType.DMA,
        ],
    )
    return fn(dst_p)[0].reshape(nc, N_PAD, d)


_BN = 2000


def _tc_layer1(sums1, cnts, x, w1lt, b1l, w1rt, w2lt):
    n = x.shape[0]
    grid = n // _BN
    nc = sums1.shape[0]

    def body(sums_ref, cnts_ref, x_ref, w1lt_ref, b1l_ref, w1rt_ref,
             w2lt_ref, h_ref, z_ref):
        s = sums_ref[0]
        c = cnts_ref[0, :, 0:1]
        for i in range(1, nc):
            s = s + sums_ref[i]
            c = c + cnts_ref[i, :, 0:1]
        mean = s / jnp.maximum(c, 1.0)
        o = (jnp.dot(mean, w1lt_ref[...], preferred_element_type=jnp.float32)
             + b1l_ref[...]
             + jnp.dot(x_ref[...], w1rt_ref[...],
                       preferred_element_type=jnp.float32))
        nrm = jnp.sqrt(jnp.sum(o * o, axis=1, keepdims=True))
        o = o / jnp.maximum(nrm, 1e-12)
        hb = jnp.maximum(o, 0.0)
        h_ref[...] = hb
        z_ref[...] = jnp.dot(hb, w2lt_ref[...],
                             preferred_element_type=jnp.float32)

    return pl.pallas_call(
        body,
        grid=(grid,),
        in_specs=[
            pl.BlockSpec((nc, _BN, D_IN), lambda i: (0, i, 0)),
            pl.BlockSpec((nc, _BN, 128), lambda i: (0, i, 0)),
            pl.BlockSpec((_BN, D_IN), lambda i: (i, 0)),
            pl.BlockSpec((D_IN, D_HID), lambda i: (0, 0)),
            pl.BlockSpec((1, D_HID), lambda i: (0, 0)),
            pl.BlockSpec((D_IN, D_HID), lambda i: (0, 0)),
            pl.BlockSpec((D_HID, D_OUT), lambda i: (0, 0)),
        ],
        out_specs=[
            pl.BlockSpec((_BN, D_HID), lambda i: (i, 0)),
            pl.BlockSpec((_BN, D_OUT), lambda i: (i, 0)),
        ],
        out_shape=[
            jax.ShapeDtypeStruct((n, D_HID), jnp.float32),
            jax.ShapeDtypeStruct((n, D_OUT), jnp.float32),
        ],
    )(sums1, cnts, x, w1lt, b1l, w1rt, w2lt)


def _tc_layer2(sums2, cnts, h, w2rt, b2l):
    n = h.shape[0]
    grid = n // _BN
    nc = sums2.shape[0]

    def body(sums_ref, cnts_ref, h_ref, w2rt_ref, b2l_ref, o_ref):
        s = sums_ref[0]
        c = cnts_ref[0, :, 0:1]
        for i in range(1, nc):
            s = s + sums_ref[i]
            c = c + cnts_ref[i, :, 0:1]
        mean = s / jnp.maximum(c, 1.0)
        o = (mean + b2l_ref[...]
             + jnp.dot(h_ref[...], w2rt_ref[...],
                       preferred_element_type=jnp.float32))
        nrm = jnp.sqrt(jnp.sum(o * o, axis=1, keepdims=True))
        o_ref[...] = o / jnp.maximum(nrm, 1e-12)

    return pl.pallas_call(
        body,
        grid=(grid,),
        in_specs=[
            pl.BlockSpec((nc, _BN, D_OUT), lambda i: (0, i, 0)),
            pl.BlockSpec((nc, _BN, 128), lambda i: (0, i, 0)),
            pl.BlockSpec((_BN, D_HID), lambda i: (i, 0)),
            pl.BlockSpec((D_HID, D_OUT), lambda i: (0, 0)),
            pl.BlockSpec((1, D_OUT), lambda i: (0, 0)),
        ],
        out_specs=pl.BlockSpec((_BN, D_OUT), lambda i: (i, 0)),
        out_shape=jax.ShapeDtypeStruct((n, D_OUT), jnp.float32),
    )(sums2, cnts, h, w2rt, b2l)


def kernel(x, edge_index, W1l, b1l, W1r, W2l, b2l, W2r):
    e = edge_index.shape[1]
    info = plsc.get_sparse_core_info()
    nw = info.num_cores * info.num_subcores
    step = nw * CHUNK
    e_pad = ((e + step - 1) // step) * step
    src = edge_index[0].astype(jnp.int32)
    dst = edge_index[1].astype(jnp.int32)
    pad = e_pad - e
    if pad:
        src = jnp.concatenate([src, jnp.zeros((pad,), jnp.int32)])
        dst = jnp.concatenate([dst, jnp.full((pad,), N_NODES, jnp.int32)])

    cnts = _seg_cnt_sc(dst)
    sums1 = _seg_sum_sc(x, src, dst)
    h, z = _tc_layer1(sums1, cnts, x, W1l.T, b1l.reshape(1, -1), W1r.T, W2l.T)
    sums2 = _seg_sum_sc(z, src, dst)
    out = _tc_layer2(sums2, cnts, h, W2r.T, b2l.reshape(1, -1))
    return out
